# baseline (speedup 1.0000x reference)
import functools

import jax
import jax.numpy as jnp
from jax import lax
from jax.experimental import pallas as pl
from jax.experimental.pallas import tpu as pltpu
from jax.experimental.pallas import tpu_sc as plsc

_T = 32768
_E = 64
_NC = 2
_NS = 16
_L = 16
_NW = _NC * _NS

_SCT = 8192
_TCT = _T - _SCT
_TPW = _SCT // _NW
_NG = _TPW // _L

_mesh = plsc.VectorSubcoreMesh(core_axis_name="c", subcore_axis_name="s")


@functools.partial(
    pl.kernel,
    out_type=(
        jax.ShapeDtypeStruct((_SCT,), jnp.int32),
        jax.ShapeDtypeStruct((_SCT,), jnp.int32),
    ),
    mesh=_mesh,
    compiler_params=pltpu.CompilerParams(
        needs_layout_passes=False,
        use_tc_tiling_on_sc=True,
    ),
    scratch_types=[
        pltpu.VMEM((_E, _TPW), jnp.float32),
        pltpu.VMEM((_TPW,), jnp.int32),
        pltpu.VMEM((_TPW,), jnp.int32),
    ],
)
def _route_mask_sc(routet_hbm, nr_hbm, r_hbm, buf, nr_buf, r_buf):
    wid = lax.axis_index("s") * _NC + lax.axis_index("c")
    tbase = wid * _TPW
    pltpu.sync_copy(routet_hbm.at[:, pl.ds(tbase, _TPW)], buf)

    @plsc.parallel_loop(0, _NG)
    def _grp(g):
        col = g * _L
        c0 = buf[0, pl.ds(col, _L)]

        def _echunk(eb, acc):
            eb8 = eb * 8
            for k in range(8):
                acc[k] = jnp.maximum(acc[k], buf[eb8 + k, pl.ds(col, _L)])
            return acc

        acc = lax.fori_loop(1, _E // 8,
                            _echunk, [buf[k, pl.ds(col, _L)]
                                      for k in range(8)])
        m = acc[0]
        for k in range(1, 8):
            m = jnp.maximum(m, acc[k])
        second = jnp.where(m > c0, 1, 0).astype(jnp.int32)
        r_buf[pl.ds(col, _L)] = second
        nr_buf[pl.ds(col, _L)] = 1 - second

    pltpu.sync_copy(nr_buf, nr_hbm.at[pl.ds(tbase, _TPW)])
    pltpu.sync_copy(r_buf, r_hbm.at[pl.ds(tbase, _TPW)])


_TCB = 2048


def _route_mask_tc_body(x_ref, nr_ref, r_ref):
    x = x_ref[...]
    m = jnp.max(x, axis=0)
    second = jnp.where(m > x[0, :], 1, 0).astype(jnp.int32)
    r_ref[...] = second
    nr_ref[...] = 1 - second


_route_mask_tc = pl.pallas_call(
    _route_mask_tc_body,
    grid=(_TCT // _TCB,),
    in_specs=[pl.BlockSpec((_E, _TCB), lambda j: (0, j + _SCT // _TCB))],
    out_specs=[pl.BlockSpec((_TCB,), lambda j: (j,)),
               pl.BlockSpec((_TCB,), lambda j: (j,))],
    out_shape=(jax.ShapeDtypeStruct((_TCT,), jnp.int32),
               jax.ShapeDtypeStruct((_TCT,), jnp.int32)),
)


def kernel(route, skip_dim):
    routet = route.T
    nr_sc, r_sc = _route_mask_sc(routet)
    nr_tc, r_tc = _route_mask_tc(routet)
    nr = jnp.concatenate([nr_sc, nr_tc])
    r = jnp.concatenate([r_sc, r_tc])
    cond = skip_dim == 1
    first = jnp.where(cond, nr, r).astype(jnp.bool_)
    second = jnp.where(cond, r, nr).astype(jnp.bool_)
    return (first, second)


# speedup vs baseline: 1.1839x; 1.1839x over previous
import functools

import jax
import jax.numpy as jnp
from jax import lax
from jax.experimental import pallas as pl
from jax.experimental.pallas import tpu as pltpu
from jax.experimental.pallas import tpu_sc as plsc

_T = 32768
_E = 64
_NC = 2
_NS = 16
_L = 16
_NW = _NC * _NS

_SCT = 8192
_TCT = _T - _SCT
_TPW = _SCT // _NW
_NG = _TPW // _L

_mesh = plsc.VectorSubcoreMesh(core_axis_name="c", subcore_axis_name="s")


@functools.partial(
    pl.kernel,
    out_type=jax.ShapeDtypeStruct((_SCT,), jnp.int32),
    mesh=_mesh,
    compiler_params=pltpu.CompilerParams(
        needs_layout_passes=False,
        use_tc_tiling_on_sc=True,
    ),
    scratch_types=[
        pltpu.VMEM((_E, _TPW), jnp.float32),
        pltpu.VMEM((_TPW,), jnp.int32),
    ],
)
def _route_mask_sc(routet_hbm, r_hbm, buf, r_buf):
    wid = lax.axis_index("s") * _NC + lax.axis_index("c")
    tbase = wid * _TPW
    pltpu.sync_copy(routet_hbm.at[:, pl.ds(tbase, _TPW)], buf)

    @plsc.parallel_loop(0, _NG)
    def _grp(g):
        col = g * _L
        c0 = buf[0, pl.ds(col, _L)]

        def _echunk(eb, acc):
            eb8 = eb * 8
            for k in range(8):
                acc[k] = jnp.maximum(acc[k], buf[eb8 + k, pl.ds(col, _L)])
            return acc

        acc = lax.fori_loop(1, _E // 8,
                            _echunk, [buf[k, pl.ds(col, _L)]
                                      for k in range(8)])
        m = acc[0]
        for k in range(1, 8):
            m = jnp.maximum(m, acc[k])
        r_buf[pl.ds(col, _L)] = jnp.where(m > c0, 1, 0).astype(jnp.int32)

    pltpu.sync_copy(r_buf, r_hbm.at[pl.ds(tbase, _TPW)])


_TCB = 4096


def _route_mask_tc_body(x_ref, r_ref):
    x = x_ref[...]
    m = jnp.max(x, axis=0)
    r_ref[...] = jnp.where(m > x[0, :], 1, 0).astype(jnp.int32)


_route_mask_tc = pl.pallas_call(
    _route_mask_tc_body,
    grid=(_TCT // _TCB,),
    in_specs=[pl.BlockSpec((_E, _TCB), lambda j: (0, j + _SCT // _TCB))],
    out_specs=pl.BlockSpec((_TCB,), lambda j: (j,)),
    out_shape=jax.ShapeDtypeStruct((_TCT,), jnp.int32),
)


def kernel(route, skip_dim):
    routet = route.T
    r_sc = _route_mask_sc(routet)
    r_tc = _route_mask_tc(routet)
    r = jnp.concatenate([r_sc, r_tc])
    c = (skip_dim == 1).astype(jnp.int32)
    rx = r ^ c
    first = rx.astype(jnp.bool_)
    second = (1 - rx).astype(jnp.bool_)
    return (first, second)


# speedup vs baseline: 1.2342x; 1.0425x over previous
import functools

import jax
import jax.numpy as jnp
from jax import lax
from jax.experimental import pallas as pl
from jax.experimental.pallas import tpu as pltpu
from jax.experimental.pallas import tpu_sc as plsc

_T = 32768
_E = 64
_NC = 1
_NS = 16
_L = 16
_NW = _NC * _NS

_SCT = 4096
_TCT = _T - _SCT
_TPW = _SCT // _NW
_NG = _TPW // _L

_mesh = plsc.VectorSubcoreMesh(core_axis_name="c", subcore_axis_name="s", num_cores=1)


@functools.partial(
    pl.kernel,
    out_type=jax.ShapeDtypeStruct((_SCT,), jnp.int32),
    mesh=_mesh,
    compiler_params=pltpu.CompilerParams(
        needs_layout_passes=False,
        use_tc_tiling_on_sc=True,
    ),
    scratch_types=[
        pltpu.VMEM((_E, _TPW), jnp.float32),
        pltpu.VMEM((_TPW,), jnp.int32),
    ],
)
def _route_mask_sc(routet_hbm, r_hbm, buf, r_buf):
    wid = lax.axis_index("s") * _NC + lax.axis_index("c")
    tbase = wid * _TPW
    pltpu.sync_copy(routet_hbm.at[:, pl.ds(tbase, _TPW)], buf)

    @plsc.parallel_loop(0, _NG)
    def _grp(g):
        col = g * _L
        c0 = buf[0, pl.ds(col, _L)]

        def _echunk(eb, acc):
            eb8 = eb * 8
            for k in range(8):
                acc[k] = jnp.maximum(acc[k], buf[eb8 + k, pl.ds(col, _L)])
            return acc

        acc = lax.fori_loop(1, _E // 8,
                            _echunk, [buf[k, pl.ds(col, _L)]
                                      for k in range(8)])
        m = acc[0]
        for k in range(1, 8):
            m = jnp.maximum(m, acc[k])
        r_buf[pl.ds(col, _L)] = jnp.where(m > c0, 1, 0).astype(jnp.int32)

    pltpu.sync_copy(r_buf, r_hbm.at[pl.ds(tbase, _TPW)])


_TCB = 4096


def _route_mask_tc_body(x_ref, r_ref):
    x = x_ref[...]
    m = jnp.max(x, axis=0)
    r_ref[...] = jnp.where(m > x[0, :], 1, 0).astype(jnp.int32)


_route_mask_tc = pl.pallas_call(
    _route_mask_tc_body,
    grid=(_TCT // _TCB,),
    in_specs=[pl.BlockSpec((_E, _TCB), lambda j: (0, j + _SCT // _TCB))],
    out_specs=pl.BlockSpec((_TCB,), lambda j: (j,)),
    out_shape=jax.ShapeDtypeStruct((_TCT,), jnp.int32),
)


def kernel(route, skip_dim):
    routet = route.T
    r_sc = _route_mask_sc(routet)
    r_tc = _route_mask_tc(routet)
    r = jnp.concatenate([r_sc, r_tc])
    c = (skip_dim == 1).astype(jnp.int32)
    rx = r ^ c
    first = rx.astype(jnp.bool_)
    second = (1 - rx).astype(jnp.bool_)
    return (first, second)
